# baseline (speedup 1.0000x reference)
import functools

import jax
import jax.numpy as jnp
from jax import lax
from jax.experimental import pallas as pl
from jax.experimental.pallas import tpu as pltpu
from jax.experimental.pallas import tpu_sc as plsc

_NUM_TRACKS = 64
_NUM_INSTRUMENTS = 128
_EMBED_DIM = 1024

_NUM_CORES = 2
_NUM_SUBCORES = 16
_NUM_WORKERS = _NUM_CORES * _NUM_SUBCORES

_CHUNK = 32
_NBUF = 3


def _pair_table_body(track_ref, instr_ref, tids_ref, iids_ref, out_ref, pid_ref):
    for a in range(track_ref.shape[0]):
        out_ref[pl.ds(a * _NUM_INSTRUMENTS, _NUM_INSTRUMENTS), :] = (
            instr_ref[...] + track_ref[a, :][None, :]
        )
    pid_ref[...] = tids_ref[...] * _NUM_INSTRUMENTS + iids_ref[...]


def _sc_gather_body(pids_hbm, pair_hbm, out_hbm, pidx_v, rows_v, gsem, ssem):
    n_chunks = pids_hbm.shape[0] // _NUM_WORKERS
    per_worker = n_chunks * _CHUNK
    wid = lax.axis_index("s") * _NUM_CORES + lax.axis_index("c")
    base = wid * per_worker
    pltpu.sync_copy(pids_hbm.at[pl.ds(wid * n_chunks, n_chunks)], pidx_v)

    def gather(c):
        return pltpu.async_copy(
            pair_hbm.at[pidx_v.at[c]], rows_v.at[c % _NBUF], gsem
        )

    def store(c):
        return pltpu.async_copy(
            rows_v.at[c % _NBUF],
            out_hbm.at[pl.ds(base + c * _CHUNK, _CHUNK)],
            ssem,
        )

    gathers = [gather(c) for c in range(min(_NBUF, n_chunks))]
    stores = []
    for c in range(n_chunks):
        gathers[c].wait()
        stores.append(store(c))
        nxt = c + _NBUF
        if nxt < n_chunks:
            stores[nxt - _NBUF].wait()
            gathers.append(gather(nxt))
    for c in range(max(0, n_chunks - _NBUF), n_chunks):
        stores[c].wait()


def kernel(track_ids, instrument_ids, track_table, instrument_table):
    batch, seq = track_ids.shape
    n_total = batch * seq
    per_worker = n_total // _NUM_WORKERS
    n_chunks = per_worker // _CHUNK

    tids = track_ids.reshape(n_total).astype(jnp.int32)
    iids = instrument_ids.reshape(n_total).astype(jnp.int32)

    n_grid = _NUM_TRACKS // 16
    pair_table, pair_ids = pl.pallas_call(
        _pair_table_body,
        grid=(n_grid,),
        in_specs=[
            pl.BlockSpec((16, _EMBED_DIM), lambda t: (t, 0)),
            pl.BlockSpec((_NUM_INSTRUMENTS, _EMBED_DIM), lambda t: (0, 0)),
            pl.BlockSpec((n_total // n_grid,), lambda t: (t,)),
            pl.BlockSpec((n_total // n_grid,), lambda t: (t,)),
        ],
        out_specs=[
            pl.BlockSpec((16 * _NUM_INSTRUMENTS, _EMBED_DIM), lambda t: (t, 0)),
            pl.BlockSpec((n_total // n_grid,), lambda t: (t,)),
        ],
        out_shape=[
            jax.ShapeDtypeStruct(
                (_NUM_TRACKS * _NUM_INSTRUMENTS, _EMBED_DIM), jnp.float32
            ),
            jax.ShapeDtypeStruct((n_total,), jnp.int32),
        ],
    )(track_table, instrument_table, tids, iids)

    sc_gather = functools.partial(
        pl.kernel,
        out_type=jax.ShapeDtypeStruct((n_total, _EMBED_DIM), jnp.float32),
        mesh=plsc.VectorSubcoreMesh(
            core_axis_name="c", subcore_axis_name="s"
        ),
        scratch_types=[
            pltpu.VMEM((n_chunks, _CHUNK), jnp.int32),
            pltpu.VMEM((_NBUF, _CHUNK, _EMBED_DIM), jnp.float32),
            pltpu.SemaphoreType.DMA,
            pltpu.SemaphoreType.DMA,
        ],
    )(_sc_gather_body)

    out = sc_gather(pair_ids.reshape(n_total // _CHUNK, _CHUNK), pair_table)
    return out.reshape(batch, seq, _EMBED_DIM)


# speedup vs baseline: 1.0068x; 1.0068x over previous
import functools

import jax
import jax.numpy as jnp
from jax import lax
from jax.experimental import pallas as pl
from jax.experimental.pallas import tpu as pltpu
from jax.experimental.pallas import tpu_sc as plsc

_NUM_TRACKS = 64
_NUM_INSTRUMENTS = 128
_EMBED_DIM = 1024

_NUM_CORES = 2
_NUM_SUBCORES = 16
_NUM_WORKERS = _NUM_CORES * _NUM_SUBCORES

_CHUNK = 32
_NBUF = 3


def _pair_table_body(track_ref, instr_ref, tids_ref, iids_ref, out_ref, pid_ref):
    for a in range(track_ref.shape[0]):
        out_ref[pl.ds(a * _NUM_INSTRUMENTS, _NUM_INSTRUMENTS), :] = (
            instr_ref[...] + track_ref[a, :][None, :]
        )
    pid_ref[...] = tids_ref[...] * _NUM_INSTRUMENTS + iids_ref[...]


def _sc_gather_body(pids_hbm, pair_hbm, out_hbm, pidx_v, rows_v, gsem, ssem):
    n_chunks = pids_hbm.shape[0] // _NUM_WORKERS
    per_worker = n_chunks * _CHUNK
    wid = lax.axis_index("s") * _NUM_CORES + lax.axis_index("c")
    base = wid * per_worker
    pltpu.sync_copy(pids_hbm.at[pl.ds(wid * n_chunks, n_chunks)], pidx_v)

    def gather(c):
        return pltpu.async_copy(
            pair_hbm.at[pidx_v.at[c]], rows_v.at[c % _NBUF], gsem
        )

    def store(c):
        return pltpu.async_copy(
            rows_v.at[c % _NBUF],
            out_hbm.at[pl.ds(base + c * _CHUNK, _CHUNK)],
            ssem,
        )

    gathers = [gather(c) for c in range(min(_NBUF, n_chunks))]
    stores = []
    for c in range(n_chunks):
        gathers[c].wait()
        stores.append(store(c))
        nxt = c + _NBUF
        if nxt < n_chunks:
            stores[nxt - _NBUF].wait()
            gathers.append(gather(nxt))
    for c in range(max(0, n_chunks - _NBUF), n_chunks):
        stores[c].wait()


def kernel(track_ids, instrument_ids, track_table, instrument_table):
    batch, seq = track_ids.shape
    n_total = batch * seq
    per_worker = n_total // _NUM_WORKERS
    n_chunks = per_worker // _CHUNK

    tids = track_ids.reshape(n_total).astype(jnp.int32)
    iids = instrument_ids.reshape(n_total).astype(jnp.int32)

    n_grid = _NUM_TRACKS // 8
    pair_table, pair_ids = pl.pallas_call(
        _pair_table_body,
        grid=(n_grid,),
        in_specs=[
            pl.BlockSpec((8, _EMBED_DIM), lambda t: (t, 0)),
            pl.BlockSpec((_NUM_INSTRUMENTS, _EMBED_DIM), lambda t: (0, 0)),
            pl.BlockSpec((n_total // n_grid,), lambda t: (t,)),
            pl.BlockSpec((n_total // n_grid,), lambda t: (t,)),
        ],
        out_specs=[
            pl.BlockSpec((8 * _NUM_INSTRUMENTS, _EMBED_DIM), lambda t: (t, 0)),
            pl.BlockSpec((n_total // n_grid,), lambda t: (t,)),
        ],
        out_shape=[
            jax.ShapeDtypeStruct(
                (_NUM_TRACKS * _NUM_INSTRUMENTS, _EMBED_DIM), jnp.float32
            ),
            jax.ShapeDtypeStruct((n_total,), jnp.int32),
        ],
    )(track_table, instrument_table, tids, iids)

    sc_gather = functools.partial(
        pl.kernel,
        out_type=jax.ShapeDtypeStruct((n_total, _EMBED_DIM), jnp.float32),
        mesh=plsc.VectorSubcoreMesh(
            core_axis_name="c", subcore_axis_name="s"
        ),
        scratch_types=[
            pltpu.VMEM((n_chunks, _CHUNK), jnp.int32),
            pltpu.VMEM((_NBUF, _CHUNK, _EMBED_DIM), jnp.float32),
            pltpu.SemaphoreType.DMA,
            pltpu.SemaphoreType.DMA,
        ],
    )(_sc_gather_body)

    out = sc_gather(pair_ids.reshape(n_total // _CHUNK, _CHUNK), pair_table)
    return out.reshape(batch, seq, _EMBED_DIM)
